# Initial kernel scaffold; baseline (speedup 1.0000x reference)
import jax
import jax.numpy as jnp
from jax import lax
from jax.experimental import pallas as pl
from jax.experimental.pallas import tpu as pltpu
from jax.experimental.pallas import tpu_sc as plsc

G = 128
D = 28
N = 1048576

NC = 2
NS = 16
NW = NC * NS
L = 16

W = 128
PPW = N // NW
NWIN = PPW // W

CORNER_OFFS = [(dx * G + dy) * G + dz
               for dx in (0, 1) for dy in (0, 1) for dz in (0, 1)]


def _body(pos_hbm, table_hbm, out_hbm, pos_v, idx_v, wt_v, cor_v, out_v, sem):
    wid = lax.axis_index("s") * NC + lax.axis_index("c")
    lane = lax.iota(jnp.int32, 16)
    c0 = jnp.full((L,), 0, jnp.int32)
    c1 = jnp.full((L,), 1, jnp.int32)
    c2 = jnp.full((L,), 2, jnp.int32)

    @pl.loop(0, NWIN)
    def _window(win):
        base = wid * PPW + win * W
        pltpu.sync_copy(pos_hbm.at[pl.ds(base, W)], pos_v)

        @pl.loop(0, W, step=L)
        def _grp(g):
            row = lane + g
            xs = plsc.load_gather(pos_v, [row, c0]) * jnp.float32(G - 1)
            ys = plsc.load_gather(pos_v, [row, c1]) * jnp.float32(G - 1)
            zs = plsc.load_gather(pos_v, [row, c2]) * jnp.float32(G - 1)
            x0 = jnp.minimum(jnp.maximum(xs.astype(jnp.int32), 0), G - 2)
            y0 = jnp.minimum(jnp.maximum(ys.astype(jnp.int32), 0), G - 2)
            z0 = jnp.minimum(jnp.maximum(zs.astype(jnp.int32), 0), G - 2)
            fx = xs - x0.astype(jnp.float32)
            fy = ys - y0.astype(jnp.float32)
            fz = zs - z0.astype(jnp.float32)
            gx = jnp.float32(1.0) - fx
            gy = jnp.float32(1.0) - fy
            gz = jnp.float32(1.0) - fz
            flat = (x0 * G + y0) * G + z0
            wxs = (gx, fx)
            wys = (gy, fy)
            wzs = (gz, fz)
            for c in range(8):
                dx, dy, dz = (c >> 2) & 1, (c >> 1) & 1, c & 1
                idx_v[c, pl.ds(g, L)] = flat + CORNER_OFFS[c]
                wt_v[c, pl.ds(g, L)] = wxs[dx] * wys[dy] * wzs[dz]

        copies = [pltpu.make_async_copy(table_hbm.at[idx_v.at[c]],
                                        cor_v.at[c], sem)
                  for c in range(8)]
        for c in range(8):
            copies[c].start()
        for c in range(8):
            copies[c].wait()

        @pl.loop(0, W)
        def _pt(w):
            wsp = jnp.full((L,), w, jnp.int32)
            acc0 = None
            acc1 = None
            for c in range(8):
                ws = plsc.load_gather(wt_v, [jnp.full((L,), c, jnp.int32), wsp])
                r0 = cor_v[c, w, pl.ds(0, L)]
                r1 = cor_v[c, w, pl.ds(D - L, L)]
                if acc0 is None:
                    acc0 = ws * r0
                    acc1 = ws * r1
                else:
                    acc0 = acc0 + ws * r0
                    acc1 = acc1 + ws * r1
            out_v[w, pl.ds(0, L)] = acc0
            out_v[w, pl.ds(D - L, L)] = acc1

        pltpu.sync_copy(out_v, out_hbm.at[pl.ds(base, W)])


def kernel(positions, voxel_grid):
    mesh = plsc.VectorSubcoreMesh(core_axis_name="c", subcore_axis_name="s")
    f = pl.kernel(
        _body,
        out_type=jax.ShapeDtypeStruct((N, D), jnp.float32),
        mesh=mesh,
        scratch_types=[
            pltpu.VMEM((W, 3), jnp.float32),
            pltpu.VMEM((8, W), jnp.int32),
            pltpu.VMEM((8, W), jnp.float32),
            pltpu.VMEM((8, W, D), jnp.float32),
            pltpu.VMEM((W, D), jnp.float32),
            pltpu.SemaphoreType.DMA,
        ],
    )
    return f(positions, voxel_grid)


# speedup vs baseline: 2.0986x; 2.0986x over previous
import dataclasses

import jax
import jax.numpy as jnp
from jax import lax
from jax.experimental import pallas as pl
from jax.experimental.pallas import tpu as pltpu
from jax.experimental.pallas import tpu_sc as plsc

G = 128
D = 28
N = 1048576

NC = 2
NS = 16
NW = NC * NS
L = 16

DP = 32
W = 128
PPW = N // NW
NWIN = PPW // W

CORNER_OFFS = [(dx * G + dy) * G + dz
               for dx in (0, 1) for dy in (0, 1) for dz in (0, 1)]


def _body(pos_hbm, table_hbm, out_hbm, pos_v, idx_v, wt_v, cor_v, out_v, sem):
    wid = lax.axis_index("s") * NC + lax.axis_index("c")
    lane = lax.iota(jnp.int32, 16)
    c0 = jnp.full((L,), 0, jnp.int32)
    c1 = jnp.full((L,), 1, jnp.int32)
    c2 = jnp.full((L,), 2, jnp.int32)

    @pl.loop(0, NWIN)
    def _window(win):
        base = wid * PPW + win * W
        pltpu.sync_copy(pos_hbm.at[pl.ds(base, W)], pos_v)

        @pl.loop(0, W, step=L)
        def _grp(g):
            row = lane + g
            xs = plsc.load_gather(pos_v, [row, c0]) * jnp.float32(G - 1)
            ys = plsc.load_gather(pos_v, [row, c1]) * jnp.float32(G - 1)
            zs = plsc.load_gather(pos_v, [row, c2]) * jnp.float32(G - 1)
            x0 = jnp.minimum(jnp.maximum(xs.astype(jnp.int32), 0), G - 2)
            y0 = jnp.minimum(jnp.maximum(ys.astype(jnp.int32), 0), G - 2)
            z0 = jnp.minimum(jnp.maximum(zs.astype(jnp.int32), 0), G - 2)
            fx = xs - x0.astype(jnp.float32)
            fy = ys - y0.astype(jnp.float32)
            fz = zs - z0.astype(jnp.float32)
            gx = jnp.float32(1.0) - fx
            gy = jnp.float32(1.0) - fy
            gz = jnp.float32(1.0) - fz
            flat = (x0 * G + y0) * G + z0
            wxs = (gx, fx)
            wys = (gy, fy)
            wzs = (gz, fz)
            for c in range(8):
                dx, dy, dz = (c >> 2) & 1, (c >> 1) & 1, c & 1
                idx_v[c, pl.ds(g, L)] = flat + CORNER_OFFS[c]
                wt_v[c, pl.ds(g, L)] = wxs[dx] * wys[dy] * wzs[dz]

        copies = [pltpu.make_async_copy(table_hbm.at[idx_v.at[c]],
                                        cor_v.at[c], sem)
                  for c in range(8)]
        for c in range(8):
            copies[c].start()
        for c in range(8):
            copies[c].wait()

        @pl.loop(0, W)
        def _pt(w):
            wsp = jnp.full((L,), w, jnp.int32)
            acc0 = None
            acc1 = None
            for c in range(8):
                ws = plsc.load_gather(wt_v, [jnp.full((L,), c, jnp.int32), wsp])
                r0 = cor_v[c, w, pl.ds(0, L)]
                r1 = cor_v[c, w, pl.ds(D - L, L)]
                if acc0 is None:
                    acc0 = ws * r0
                    acc1 = ws * r1
                else:
                    acc0 = acc0 + ws * r0
                    acc1 = acc1 + ws * r1
            out_v[w, pl.ds(0, L)] = acc0
            out_v[w, pl.ds(D - L, L)] = acc1

        pltpu.sync_copy(out_v, out_hbm.at[pl.ds(base, W)])


def kernel(positions, voxel_grid):
    mesh = plsc.VectorSubcoreMesh(core_axis_name="c", subcore_axis_name="s")
    cp = pltpu.CompilerParams()
    for field, val in (("needs_layout_passes", False),
                       ("use_tc_tiling_on_sc", False)):
        if field in pltpu.CompilerParams.__dataclass_fields__:
            cp = dataclasses.replace(cp, **{field: val})
    f = pl.kernel(
        _body,
        compiler_params=cp,
        out_type=jax.ShapeDtypeStruct((N, D), jnp.float32),
        mesh=mesh,
        scratch_types=[
            pltpu.VMEM((W, 3), jnp.float32),
            pltpu.VMEM((8, W), jnp.int32),
            pltpu.VMEM((8, W), jnp.float32),
            pltpu.VMEM((8, W, DP), jnp.float32),
            pltpu.VMEM((W, D), jnp.float32),
            pltpu.SemaphoreType.DMA,
        ],
    )
    table = jnp.pad(voxel_grid, ((0, 0), (0, DP - D)))
    return f(positions, table)


# speedup vs baseline: 2.3808x; 1.1345x over previous
import dataclasses

import jax
import jax.numpy as jnp
from jax import lax
from jax.experimental import pallas as pl
from jax.experimental.pallas import tpu as pltpu
from jax.experimental.pallas import tpu_sc as plsc

G = 128
D = 28
N = 1048576
V = G * G * G

NC = 2
NS = 16
NW = NC * NS
L = 16

DP = 32
W = 128
PPW = N // NW
NWIN = PPW // W

RC = 512
RPW = V // NW

CORNER_OFFS = [(dx * G + dy) * G + dz
               for dx in (0, 1) for dy in (0, 1) for dz in (0, 1)]


def _repack_body(t1d_hbm, t32_hbm, in_v, out_v, sem):
    wid = lax.axis_index("s") * NC + lax.axis_index("c")
    base = wid * RPW

    @pl.loop(0, RPW, step=RC)
    def _chunk(r0):
        pltpu.sync_copy(t1d_hbm.at[pl.ds((base + r0) * D, RC * D)], in_v)

        @pl.loop(0, RC)
        def _row(r):
            out_v[r, pl.ds(0, L)] = in_v[pl.ds(r * D, L)]
            out_v[r, pl.ds(D - L, L)] = in_v[pl.ds(r * D + D - L, L)]

        pltpu.sync_copy(out_v, t32_hbm.at[pl.ds(base + r0, RC)])


def _main_body(pos_hbm, table_hbm, out_hbm, pos_v, idx_v, wt_v, cor_v, out_v,
               sem):
    wid = lax.axis_index("s") * NC + lax.axis_index("c")
    lane = lax.iota(jnp.int32, 16)
    lane3 = lane * 3

    @pl.loop(0, NWIN)
    def _window(win):
        base = wid * PPW + win * W
        pltpu.sync_copy(pos_hbm.at[pl.ds(base * 3, W * 3)], pos_v)

        @pl.loop(0, W, step=L)
        def _grp(g):
            g3 = g * 3
            xs = plsc.load_gather(pos_v, [lane3 + g3]) * jnp.float32(G - 1)
            ys = plsc.load_gather(pos_v, [lane3 + (g3 + 1)]) * jnp.float32(G - 1)
            zs = plsc.load_gather(pos_v, [lane3 + (g3 + 2)]) * jnp.float32(G - 1)
            x0 = jnp.minimum(jnp.maximum(xs.astype(jnp.int32), 0), G - 2)
            y0 = jnp.minimum(jnp.maximum(ys.astype(jnp.int32), 0), G - 2)
            z0 = jnp.minimum(jnp.maximum(zs.astype(jnp.int32), 0), G - 2)
            fx = xs - x0.astype(jnp.float32)
            fy = ys - y0.astype(jnp.float32)
            fz = zs - z0.astype(jnp.float32)
            gx = jnp.float32(1.0) - fx
            gy = jnp.float32(1.0) - fy
            gz = jnp.float32(1.0) - fz
            flat = (x0 * G + y0) * G + z0
            wxs = (gx, fx)
            wys = (gy, fy)
            wzs = (gz, fz)
            for c in range(8):
                dx, dy, dz = (c >> 2) & 1, (c >> 1) & 1, c & 1
                idx_v[c, pl.ds(g, L)] = flat + CORNER_OFFS[c]
                wt_v[c, pl.ds(g, L)] = wxs[dx] * wys[dy] * wzs[dz]

        copies = [pltpu.make_async_copy(table_hbm.at[idx_v.at[c]],
                                        cor_v.at[c], sem)
                  for c in range(8)]
        for c in range(8):
            copies[c].start()
        for c in range(8):
            copies[c].wait()

        @pl.loop(0, W)
        def _pt(w):
            wsp = jnp.full((L,), w, jnp.int32)
            acc0 = None
            acc1 = None
            for c in range(8):
                ws = plsc.load_gather(wt_v, [jnp.full((L,), c, jnp.int32), wsp])
                r0 = cor_v[c, w, pl.ds(0, L)]
                r1 = cor_v[c, w, pl.ds(D - L, L)]
                if acc0 is None:
                    acc0 = ws * r0
                    acc1 = ws * r1
                else:
                    acc0 = acc0 + ws * r0
                    acc1 = acc1 + ws * r1
            w28 = w * D
            out_v[pl.ds(w28, L)] = acc0
            out_v[pl.ds(w28 + D - L, L)] = acc1

        pltpu.sync_copy(out_v, out_hbm.at[pl.ds(base * D, W * D)])


def _make_cp():
    cp = pltpu.CompilerParams()
    for field, val in (("needs_layout_passes", False),
                       ("use_tc_tiling_on_sc", False)):
        if field in pltpu.CompilerParams.__dataclass_fields__:
            cp = dataclasses.replace(cp, **{field: val})
    return cp


def kernel(positions, voxel_grid):
    mesh = plsc.VectorSubcoreMesh(core_axis_name="c", subcore_axis_name="s")
    cp = _make_cp()

    repack = pl.kernel(
        _repack_body,
        out_type=jax.ShapeDtypeStruct((V, DP), jnp.float32),
        mesh=mesh,
        compiler_params=cp,
        scratch_types=[
            pltpu.VMEM((RC * D,), jnp.float32),
            pltpu.VMEM((RC, DP), jnp.float32),
            pltpu.SemaphoreType.DMA,
        ],
    )

    main = pl.kernel(
        _main_body,
        out_type=jax.ShapeDtypeStruct((N * D,), jnp.float32),
        mesh=mesh,
        compiler_params=cp,
        scratch_types=[
            pltpu.VMEM((W * 3,), jnp.float32),
            pltpu.VMEM((8, W), jnp.int32),
            pltpu.VMEM((8, W), jnp.float32),
            pltpu.VMEM((8, W, DP), jnp.float32),
            pltpu.VMEM((W * D,), jnp.float32),
            pltpu.SemaphoreType.DMA,
        ],
    )

    t32 = repack(voxel_grid.reshape(V * D))
    out1d = main(positions.reshape(N * 3), t32)
    return out1d.reshape(N, D)
